# baseline (speedup 1.0000x reference)
import functools

import jax
import jax.numpy as jnp
from jax import lax
from jax.experimental import pallas as pl
from jax.experimental.pallas import tpu as pltpu
from jax.experimental.pallas import tpu_sc as plsc

_GAMMA = 0.1
_STEP_SIZE = 0.01

_NC = 2
_NS = 16
_NW = _NC * _NS
_LANES = 16


_BANKS = 5


def _make_segment_partials(n, n_groups):
    per_tile = n // _NW
    chunks = per_tile // _LANES
    nbins = n_groups * _LANES
    unroll = 25 if chunks % 25 == 0 else (5 if chunks % 5 == 0 else 1)
    banks = _BANKS if unroll % _BANKS == 0 else 1
    mesh = plsc.VectorSubcoreMesh(core_axis_name="c", subcore_axis_name="s")

    @functools.partial(
        pl.kernel,
        mesh=mesh,
        compiler_params=pltpu.CompilerParams(needs_layout_passes=False),
        out_type=[
            jax.ShapeDtypeStruct((_NW, nbins), jnp.float32),
            jax.ShapeDtypeStruct((_NW, nbins), jnp.float32),
        ],
        scratch_types=[
            pltpu.VMEM((per_tile,), jnp.float32),
            pltpu.VMEM((per_tile,), jnp.int32),
            pltpu.VMEM((banks * nbins,), jnp.float32),
            pltpu.VMEM((banks * nbins,), jnp.float32),
        ],
    )
    def seg(loss_hbm, idx_hbm, sums_hbm, cnts_hbm, loss_v, idx_v, acc_v, cnt_v):
        wid = lax.axis_index("s") * _NC + lax.axis_index("c")
        base = wid * per_tile
        pltpu.sync_copy(loss_hbm.at[pl.ds(base, per_tile)], loss_v)
        pltpu.sync_copy(idx_hbm.at[pl.ds(base, per_tile)], idx_v)
        zeros = jnp.zeros((_LANES,), jnp.float32)
        for k in range(banks * n_groups):
            acc_v[pl.ds(k * _LANES, _LANES)] = zeros
            cnt_v[pl.ds(k * _LANES, _LANES)] = zeros
        ones = jnp.ones((_LANES,), jnp.float32)
        lane = lax.iota(jnp.int32, _LANES)

        def body(j, carry):
            off = j * (_LANES * unroll)
            for u in range(unroll):
                o = off + u * _LANES
                bank_off = (u % banks) * nbins
                v = loss_v[pl.ds(o, _LANES)]
                i = idx_v[pl.ds(o, _LANES)]
                a = lax.shift_left(i, 4) + (lane + bank_off)
                plsc.addupdate_scatter(acc_v, [a], v)
                plsc.addupdate_scatter(cnt_v, [a], ones)
            return carry

        lax.fori_loop(0, chunks // unroll, body, 0)
        for b in range(1, banks):
            for k in range(n_groups):
                o0 = k * _LANES
                ob = b * nbins + k * _LANES
                acc_v[pl.ds(o0, _LANES)] = (
                    acc_v[pl.ds(o0, _LANES)] + acc_v[pl.ds(ob, _LANES)])
                cnt_v[pl.ds(o0, _LANES)] = (
                    cnt_v[pl.ds(o0, _LANES)] + cnt_v[pl.ds(ob, _LANES)])
        pltpu.sync_copy(acc_v.at[pl.ds(0, nbins)], sums_hbm.at[wid])
        pltpu.sync_copy(cnt_v.at[pl.ds(0, nbins)], cnts_hbm.at[wid])

    return seg


def _epilogue_body(sums_ref, cnts_ref, adv_ref, out_ref):
    nbins = sums_ref.shape[1]
    n_groups = adv_ref.shape[1]
    s = jnp.sum(sums_ref[...], axis=0, keepdims=True)
    c = jnp.sum(cnts_ref[...], axis=0, keepdims=True)
    r = lax.shift_right_logical(
        lax.broadcasted_iota(jnp.int32, (nbins, n_groups), 0), 4)
    b = lax.broadcasted_iota(jnp.int32, (nbins, n_groups), 1)
    sel = (r == b).astype(jnp.float32)
    gsum = jax.lax.dot_general(
        s, sel, (((1,), (0,)), ((), ())),
        preferred_element_type=jnp.float32)
    gcnt = jax.lax.dot_general(
        c, sel, (((1,), (0,)), ((), ())),
        preferred_element_type=jnp.float32)
    denom = gcnt + (gcnt == 0).astype(jnp.float32)
    gl = gsum / denom
    adv = adv_ref[...] * jnp.exp(_STEP_SIZE * gl)
    num = jnp.sum(gl * adv, axis=1, keepdims=True)
    den = jnp.sum(adv, axis=1, keepdims=True)
    out_ref[...] = num / den


def kernel(loss, group_idx, adv_probs, exp_avg_loss, group_counts, adj):
    n = loss.shape[0]
    n_groups = adv_probs.shape[0]
    seg = _make_segment_partials(n, n_groups)
    sums, cnts = seg(loss, group_idx.astype(jnp.int32))
    out = pl.pallas_call(
        _epilogue_body,
        out_shape=jax.ShapeDtypeStruct((1, 1), jnp.float32),
    )(sums, cnts, adv_probs.reshape(1, n_groups))
    return out[0, 0]


# speedup vs baseline: 1.0263x; 1.0263x over previous
import functools

import jax
import jax.numpy as jnp
from jax import lax
from jax.experimental import pallas as pl
from jax.experimental.pallas import tpu as pltpu
from jax.experimental.pallas import tpu_sc as plsc

_GAMMA = 0.1
_STEP_SIZE = 0.01

_NC = 2
_NS = 16
_NW = _NC * _NS
_LANES = 16


_BANKS = 1


def _make_segment_partials(n, n_groups):
    per_tile = n // _NW
    chunks = per_tile // _LANES
    nbins = n_groups * _LANES
    unroll = 25 if chunks % 25 == 0 else (5 if chunks % 5 == 0 else 1)
    banks = _BANKS if unroll % _BANKS == 0 else 1
    mesh = plsc.VectorSubcoreMesh(core_axis_name="c", subcore_axis_name="s")

    @functools.partial(
        pl.kernel,
        mesh=mesh,
        compiler_params=pltpu.CompilerParams(needs_layout_passes=False),
        out_type=[
            jax.ShapeDtypeStruct((_NW, nbins), jnp.float32),
            jax.ShapeDtypeStruct((_NW, nbins), jnp.float32),
        ],
        scratch_types=[
            pltpu.VMEM((per_tile,), jnp.float32),
            pltpu.VMEM((per_tile,), jnp.int32),
            pltpu.VMEM((banks * nbins,), jnp.float32),
            pltpu.VMEM((banks * nbins,), jnp.float32),
        ],
    )
    def seg(loss_hbm, idx_hbm, sums_hbm, cnts_hbm, loss_v, idx_v, acc_v, cnt_v):
        wid = lax.axis_index("s") * _NC + lax.axis_index("c")
        base = wid * per_tile
        pltpu.sync_copy(loss_hbm.at[pl.ds(base, per_tile)], loss_v)
        pltpu.sync_copy(idx_hbm.at[pl.ds(base, per_tile)], idx_v)
        zeros = jnp.zeros((_LANES,), jnp.float32)
        for k in range(banks * n_groups):
            acc_v[pl.ds(k * _LANES, _LANES)] = zeros
            cnt_v[pl.ds(k * _LANES, _LANES)] = zeros
        ones = jnp.ones((_LANES,), jnp.float32)
        lane = lax.iota(jnp.int32, _LANES)

        def body(j, carry):
            off = j * (_LANES * unroll)
            for u in range(unroll):
                o = off + u * _LANES
                bank_off = (u % banks) * nbins
                v = loss_v[pl.ds(o, _LANES)]
                i = idx_v[pl.ds(o, _LANES)]
                a = lax.shift_left(i, 4) + (lane + bank_off)
                plsc.addupdate_scatter(acc_v, [a], v)
                plsc.addupdate_scatter(cnt_v, [a], ones)
            return carry

        lax.fori_loop(0, chunks // unroll, body, 0)
        for b in range(1, banks):
            for k in range(n_groups):
                o0 = k * _LANES
                ob = b * nbins + k * _LANES
                acc_v[pl.ds(o0, _LANES)] = (
                    acc_v[pl.ds(o0, _LANES)] + acc_v[pl.ds(ob, _LANES)])
                cnt_v[pl.ds(o0, _LANES)] = (
                    cnt_v[pl.ds(o0, _LANES)] + cnt_v[pl.ds(ob, _LANES)])
        pltpu.sync_copy(acc_v.at[pl.ds(0, nbins)], sums_hbm.at[wid])
        pltpu.sync_copy(cnt_v.at[pl.ds(0, nbins)], cnts_hbm.at[wid])

    return seg


def _epilogue_body(sums_ref, cnts_ref, adv_ref, out_ref):
    nbins = sums_ref.shape[1]
    n_groups = adv_ref.shape[1]
    s = jnp.sum(sums_ref[...], axis=0, keepdims=True)
    c = jnp.sum(cnts_ref[...], axis=0, keepdims=True)
    r = lax.shift_right_logical(
        lax.broadcasted_iota(jnp.int32, (nbins, n_groups), 0), 4)
    b = lax.broadcasted_iota(jnp.int32, (nbins, n_groups), 1)
    sel = (r == b).astype(jnp.float32)
    gsum = jax.lax.dot_general(
        s, sel, (((1,), (0,)), ((), ())),
        preferred_element_type=jnp.float32)
    gcnt = jax.lax.dot_general(
        c, sel, (((1,), (0,)), ((), ())),
        preferred_element_type=jnp.float32)
    denom = gcnt + (gcnt == 0).astype(jnp.float32)
    gl = gsum / denom
    adv = adv_ref[...] * jnp.exp(_STEP_SIZE * gl)
    num = jnp.sum(gl * adv, axis=1, keepdims=True)
    den = jnp.sum(adv, axis=1, keepdims=True)
    out_ref[...] = num / den


def kernel(loss, group_idx, adv_probs, exp_avg_loss, group_counts, adj):
    n = loss.shape[0]
    n_groups = adv_probs.shape[0]
    seg = _make_segment_partials(n, n_groups)
    sums, cnts = seg(loss, group_idx.astype(jnp.int32))
    out = pl.pallas_call(
        _epilogue_body,
        out_shape=jax.ShapeDtypeStruct((1, 1), jnp.float32),
    )(sums, cnts, adv_probs.reshape(1, n_groups))
    return out[0, 0]
